# baseline (speedup 1.0000x reference)
import functools

import jax
import jax.numpy as jnp
from jax import lax
from jax.experimental import pallas as pl
from jax.experimental.pallas import tpu as pltpu
from jax.experimental.pallas import tpu_sc as plsc

B = 128
N_LAYERS = 12
KEY_D = 768
EMB_D = 768
POOL = 512
NUM_PROMPTS = 8
TOP_K = 5

ROW_D = NUM_PROMPTS * EMB_D
ROWS = N_LAYERS * B * TOP_K
NUM_CORES = 2
NUM_SUBCORES = 16
NW = NUM_CORES * NUM_SUBCORES
RPW = ROWS // NW
CHUNK = 10
NCHUNK = RPW // CHUNK
NBUF = 2


def _topk_body(q_ref, k_ref, idx_ref):
    s = jax.lax.dot_general(
        q_ref[0], k_ref[0], (((1,), (1,)), ((), ())),
        preferred_element_type=jnp.float32)
    col = lax.broadcasted_iota(jnp.int32, (B, POOL), 1)
    neg = jnp.float32(-jnp.inf)
    outs = []
    for _ in range(TOP_K):
        m = jnp.max(s, axis=1, keepdims=True)
        cand = jnp.where(s == m, col, POOL)
        idx = jnp.min(cand, axis=1, keepdims=True)
        outs.append(idx)
        s = jnp.where(col == idx, neg, s)
    base = pl.program_id(0) * POOL
    idx_ref[0] = jnp.concatenate(outs, axis=1) + base


def _topk_indices(qn, kn):
    return pl.pallas_call(
        _topk_body,
        grid=(N_LAYERS,),
        in_specs=[
            pl.BlockSpec((1, B, KEY_D), lambda l: (l, 0, 0)),
            pl.BlockSpec((1, POOL, KEY_D), lambda l: (l, 0, 0)),
        ],
        out_specs=pl.BlockSpec((1, B, TOP_K), lambda l: (l, 0, 0)),
        out_shape=jax.ShapeDtypeStruct((N_LAYERS, B, TOP_K), jnp.int32),
    )(qn, kn)


def _sc_gather_body(table_hbm, idx_hbm, out_hbm, idx_v, buf0, buf1,
                    gsem0, gsem1, ssem0, ssem1):
    wid = lax.axis_index("s") * NUM_CORES + lax.axis_index("c")
    pltpu.sync_copy(idx_hbm.at[wid], idx_v)
    base = wid * RPW

    bufs = (buf0, buf1)
    gsems = (gsem0, gsem1)
    ssems = (ssem0, ssem1)

    def ostart(c):
        return out_hbm.at[pl.ds(base + c * CHUNK, CHUNK)]

    for c in range(NBUF - 1):
        pltpu.async_copy(table_hbm.at[idx_v.at[c]], bufs[c], gsems[c])

    def chunk(c, carry):
        for p in range(NBUF):
            @pl.when(c % NBUF == p)
            def _():
                pltpu.make_async_copy(table_hbm.at[idx_v.at[c]], bufs[p],
                                      gsems[p]).wait()
                pltpu.async_copy(bufs[p], ostart(c), ssems[p])

        nxt = c + NBUF - 1
        for p in range(NBUF):
            @pl.when(jnp.logical_and(nxt < NCHUNK, nxt % NBUF == p))
            def _():
                @pl.when(nxt >= NBUF)
                def _():
                    pltpu.make_async_copy(bufs[p], ostart(nxt - NBUF),
                                          ssems[p]).wait()
                pltpu.async_copy(table_hbm.at[idx_v.at[nxt]], bufs[p], gsems[p])

        return carry

    lax.fori_loop(0, NCHUNK, chunk, 0)
    for last in range(NCHUNK - NBUF, NCHUNK):
        pltpu.make_async_copy(bufs[last % NBUF], ostart(last),
                              ssems[last % NBUF]).wait()


@functools.lru_cache(maxsize=1)
def _sc_gather():
    return functools.partial(
        pl.kernel,
        mesh=plsc.VectorSubcoreMesh(core_axis_name="c", subcore_axis_name="s"),
        out_type=jax.ShapeDtypeStruct((ROWS, NUM_PROMPTS, EMB_D), jnp.float32),
        scratch_types=(
            [pltpu.VMEM((NCHUNK, CHUNK), jnp.int32)]
            + [pltpu.VMEM((CHUNK, NUM_PROMPTS, EMB_D), jnp.float32)] * NBUF
            + [pltpu.SemaphoreType.DMA] * (2 * NBUF)
        ),
        compiler_params=pltpu.CompilerParams(use_tc_tiling_on_sc=True),
    )(_sc_gather_body)


def _normalize(x):
    n = jnp.linalg.norm(x, axis=-1, keepdims=True)
    return x / jnp.maximum(n, 1e-12)


def kernel(x_query, e_k, e_p, vis_mark):
    xq = jnp.swapaxes(x_query, 0, 1)
    qn = _normalize(xq)
    kn = _normalize(e_k)
    idx = _topk_indices(qn, kn)
    flat_idx = idx.reshape(NW, NCHUNK, CHUNK)
    table = e_p.reshape(N_LAYERS * POOL, NUM_PROMPTS, EMB_D)
    out = _sc_gather()(table, flat_idx)
    out = out.reshape(N_LAYERS, B, TOP_K * NUM_PROMPTS, EMB_D)
    p_loss = jnp.zeros((), jnp.float32)
    return (out, p_loss)


# speedup vs baseline: 1.0578x; 1.0578x over previous
import functools

import jax
import jax.numpy as jnp
from jax import lax
from jax.experimental import pallas as pl
from jax.experimental.pallas import tpu as pltpu
from jax.experimental.pallas import tpu_sc as plsc

B = 128
N_LAYERS = 12
KEY_D = 768
EMB_D = 768
POOL = 512
NUM_PROMPTS = 8
TOP_K = 5

ROW_D = NUM_PROMPTS * EMB_D
ROWS = N_LAYERS * B * TOP_K
NUM_CORES = 2
NUM_SUBCORES = 16
NW = NUM_CORES * NUM_SUBCORES
RPW = ROWS // NW
CHUNK = 10
NCHUNK = RPW // CHUNK
NBUF = 2


def _topk_body(q_ref, nq_ref, k_ref, nk_ref, idx_ref):
    qn = q_ref[0] / nq_ref[0]
    kn = k_ref[0] / nk_ref[0]
    s = jax.lax.dot_general(
        qn, kn, (((1,), (1,)), ((), ())),
        preferred_element_type=jnp.float32)
    col = lax.broadcasted_iota(jnp.int32, (B, POOL), 1)
    neg = jnp.float32(-jnp.inf)
    outs = []
    for _ in range(TOP_K):
        m = jnp.max(s, axis=1, keepdims=True)
        cand = jnp.where(s == m, col, POOL)
        idx = jnp.min(cand, axis=1, keepdims=True)
        outs.append(idx)
        s = jnp.where(col == idx, neg, s)
    base = pl.program_id(0) * POOL
    idx_ref[0] = jnp.concatenate(outs, axis=1) + base


def _topk_indices(xq, nq, e_k, nk):
    return pl.pallas_call(
        _topk_body,
        grid=(N_LAYERS,),
        in_specs=[
            pl.BlockSpec((1, B, KEY_D), lambda l: (l, 0, 0)),
            pl.BlockSpec((1, B, 1), lambda l: (l, 0, 0)),
            pl.BlockSpec((1, POOL, KEY_D), lambda l: (l, 0, 0)),
            pl.BlockSpec((1, POOL, 1), lambda l: (l, 0, 0)),
        ],
        out_specs=pl.BlockSpec((1, B, TOP_K), lambda l: (l, 0, 0)),
        out_shape=jax.ShapeDtypeStruct((N_LAYERS, B, TOP_K), jnp.int32),
    )(xq, nq, e_k, nk)


def _sc_gather_body(table_hbm, idx_hbm, out_hbm, idx_v, buf0, buf1,
                    gsem0, gsem1, ssem0, ssem1):
    wid = lax.axis_index("s") * NUM_CORES + lax.axis_index("c")
    pltpu.sync_copy(idx_hbm.at[wid], idx_v)
    base = wid * RPW

    bufs = (buf0, buf1)
    gsems = (gsem0, gsem1)
    ssems = (ssem0, ssem1)

    def ostart(c):
        return out_hbm.at[pl.ds(base + c * CHUNK, CHUNK)]

    for c in range(NBUF - 1):
        pltpu.async_copy(table_hbm.at[idx_v.at[c]], bufs[c], gsems[c])

    def chunk(c, carry):
        for p in range(NBUF):
            @pl.when(c % NBUF == p)
            def _():
                pltpu.make_async_copy(table_hbm.at[idx_v.at[c]], bufs[p],
                                      gsems[p]).wait()
                pltpu.async_copy(bufs[p], ostart(c), ssems[p])

        nxt = c + NBUF - 1
        for p in range(NBUF):
            @pl.when(jnp.logical_and(nxt < NCHUNK, nxt % NBUF == p))
            def _():
                @pl.when(nxt >= NBUF)
                def _():
                    pltpu.make_async_copy(bufs[p], ostart(nxt - NBUF),
                                          ssems[p]).wait()
                pltpu.async_copy(table_hbm.at[idx_v.at[nxt]], bufs[p], gsems[p])

        return carry

    lax.fori_loop(0, NCHUNK, chunk, 0)
    for last in range(NCHUNK - NBUF, NCHUNK):
        pltpu.make_async_copy(bufs[last % NBUF], ostart(last),
                              ssems[last % NBUF]).wait()


@functools.lru_cache(maxsize=1)
def _sc_gather():
    return functools.partial(
        pl.kernel,
        mesh=plsc.VectorSubcoreMesh(core_axis_name="c", subcore_axis_name="s"),
        out_type=jax.ShapeDtypeStruct((ROWS, NUM_PROMPTS, EMB_D), jnp.float32),
        scratch_types=(
            [pltpu.VMEM((NCHUNK, CHUNK), jnp.int32)]
            + [pltpu.VMEM((CHUNK, NUM_PROMPTS, EMB_D), jnp.float32)] * NBUF
            + [pltpu.SemaphoreType.DMA] * (2 * NBUF)
        ),
        compiler_params=pltpu.CompilerParams(use_tc_tiling_on_sc=True),
    )(_sc_gather_body)


def kernel(x_query, e_k, e_p, vis_mark):
    xq = jnp.swapaxes(x_query, 0, 1)
    nq = jnp.maximum(jnp.linalg.norm(xq, axis=-1, keepdims=True), 1e-12)
    nk = jnp.maximum(jnp.linalg.norm(e_k, axis=-1, keepdims=True), 1e-12)
    idx = _topk_indices(xq, nq, e_k, nk)
    flat_idx = idx.reshape(NW, NCHUNK, CHUNK)
    table = e_p.reshape(N_LAYERS * POOL, NUM_PROMPTS, EMB_D)
    out = _sc_gather()(table, flat_idx)
    out = out.reshape(N_LAYERS, B, TOP_K * NUM_PROMPTS, EMB_D)
    p_loss = jnp.zeros((), jnp.float32)
    return (out, p_loss)


# speedup vs baseline: 1.1029x; 1.0427x over previous
import functools

import jax
import jax.numpy as jnp
from jax import lax
from jax.experimental import pallas as pl
from jax.experimental.pallas import tpu as pltpu
from jax.experimental.pallas import tpu_sc as plsc

B = 128
N_LAYERS = 12
KEY_D = 768
EMB_D = 768
POOL = 512
NUM_PROMPTS = 8
TOP_K = 5

ROW_D = NUM_PROMPTS * EMB_D
ROWS = N_LAYERS * B * TOP_K
NUM_CORES = 2
NUM_SUBCORES = 16
NW = NUM_CORES * NUM_SUBCORES
RPW = ROWS // NW
CHUNK = 10
NCHUNK = RPW // CHUNK
NBUF = 2
LPS = 2


def _topk_body(q_ref, nq_ref, k_ref, nk_ref, idx_ref):
    col = lax.broadcasted_iota(jnp.int32, (B, POOL), 1)
    neg = jnp.float32(-jnp.inf)
    for j in range(LPS):
        qn = q_ref[j] / nq_ref[j]
        kn = k_ref[j] / nk_ref[j]
        s = jax.lax.dot_general(
            qn, kn, (((1,), (1,)), ((), ())),
            preferred_element_type=jnp.float32)
        outs = []
        for _ in range(TOP_K):
            m = jnp.max(s, axis=1, keepdims=True)
            cand = jnp.where(s == m, col, POOL)
            idx = jnp.min(cand, axis=1, keepdims=True)
            outs.append(idx)
            s = jnp.where(col == idx, neg, s)
        base = (pl.program_id(0) * LPS + j) * POOL
        idx_ref[j] = jnp.concatenate(outs, axis=1) + base


def _topk_indices(xq, nq, e_k, nk):
    return pl.pallas_call(
        _topk_body,
        grid=(N_LAYERS // LPS,),
        in_specs=[
            pl.BlockSpec((LPS, B, KEY_D), lambda l: (l, 0, 0)),
            pl.BlockSpec((LPS, B, 1), lambda l: (l, 0, 0)),
            pl.BlockSpec((LPS, POOL, KEY_D), lambda l: (l, 0, 0)),
            pl.BlockSpec((LPS, POOL, 1), lambda l: (l, 0, 0)),
        ],
        out_specs=pl.BlockSpec((LPS, B, TOP_K), lambda l: (l, 0, 0)),
        out_shape=jax.ShapeDtypeStruct((N_LAYERS, B, TOP_K), jnp.int32),
    )(xq, nq, e_k, nk)


def _sc_gather_body(table_hbm, idx_hbm, out_hbm, idx_v, buf0, buf1,
                    gsem0, gsem1, ssem0, ssem1):
    wid = lax.axis_index("s") * NUM_CORES + lax.axis_index("c")
    pltpu.sync_copy(idx_hbm.at[wid], idx_v)
    base = wid * RPW

    bufs = (buf0, buf1)
    gsems = (gsem0, gsem1)
    ssems = (ssem0, ssem1)

    def ostart(c):
        return out_hbm.at[pl.ds(base + c * CHUNK, CHUNK)]

    for c in range(NBUF - 1):
        pltpu.async_copy(table_hbm.at[idx_v.at[c]], bufs[c], gsems[c])

    def chunk(c, carry):
        for p in range(NBUF):
            @pl.when(c % NBUF == p)
            def _():
                pltpu.make_async_copy(table_hbm.at[idx_v.at[c]], bufs[p],
                                      gsems[p]).wait()
                pltpu.async_copy(bufs[p], ostart(c), ssems[p])

        nxt = c + NBUF - 1
        for p in range(NBUF):
            @pl.when(jnp.logical_and(nxt < NCHUNK, nxt % NBUF == p))
            def _():
                @pl.when(nxt >= NBUF)
                def _():
                    pltpu.make_async_copy(bufs[p], ostart(nxt - NBUF),
                                          ssems[p]).wait()
                pltpu.async_copy(table_hbm.at[idx_v.at[nxt]], bufs[p], gsems[p])

        return carry

    lax.fori_loop(0, NCHUNK, chunk, 0)
    for last in range(NCHUNK - NBUF, NCHUNK):
        pltpu.make_async_copy(bufs[last % NBUF], ostart(last),
                              ssems[last % NBUF]).wait()


@functools.lru_cache(maxsize=1)
def _sc_gather():
    return functools.partial(
        pl.kernel,
        mesh=plsc.VectorSubcoreMesh(core_axis_name="c", subcore_axis_name="s"),
        out_type=jax.ShapeDtypeStruct((ROWS, NUM_PROMPTS, EMB_D), jnp.float32),
        scratch_types=(
            [pltpu.VMEM((NCHUNK, CHUNK), jnp.int32)]
            + [pltpu.VMEM((CHUNK, NUM_PROMPTS, EMB_D), jnp.float32)] * NBUF
            + [pltpu.SemaphoreType.DMA] * (2 * NBUF)
        ),
        compiler_params=pltpu.CompilerParams(use_tc_tiling_on_sc=True),
    )(_sc_gather_body)


def kernel(x_query, e_k, e_p, vis_mark):
    xq = jnp.swapaxes(x_query, 0, 1)
    nq = jnp.maximum(jnp.linalg.norm(xq, axis=-1, keepdims=True), 1e-12)
    nk = jnp.maximum(jnp.linalg.norm(e_k, axis=-1, keepdims=True), 1e-12)
    idx = _topk_indices(xq, nq, e_k, nk)
    flat_idx = idx.reshape(NW, NCHUNK, CHUNK)
    table = e_p.reshape(N_LAYERS * POOL, NUM_PROMPTS, EMB_D)
    out = _sc_gather()(table, flat_idx)
    out = out.reshape(N_LAYERS, B, TOP_K * NUM_PROMPTS, EMB_D)
    p_loss = jnp.zeros((), jnp.float32)
    return (out, p_loss)


# speedup vs baseline: 1.1122x; 1.0084x over previous
import functools

import jax
import jax.numpy as jnp
from jax import lax
from jax.experimental import pallas as pl
from jax.experimental.pallas import tpu as pltpu
from jax.experimental.pallas import tpu_sc as plsc

B = 128
N_LAYERS = 12
KEY_D = 768
EMB_D = 768
POOL = 512
NUM_PROMPTS = 8
TOP_K = 5

ROW_D = NUM_PROMPTS * EMB_D
ROWS = N_LAYERS * B * TOP_K
NUM_CORES = 2
NUM_SUBCORES = 16
NW = NUM_CORES * NUM_SUBCORES
RPW = ROWS // NW
CHUNK = 10
NCHUNK = RPW // CHUNK
NBUF = 2
LPS = 4


def _topk_body(q_ref, nq_ref, k_ref, nk_ref, idx_ref):
    col = lax.broadcasted_iota(jnp.int32, (B, POOL), 1)
    neg = jnp.float32(-jnp.inf)
    for j in range(LPS):
        qn = q_ref[j] / nq_ref[j]
        kn = k_ref[j] / nk_ref[j]
        s = jax.lax.dot_general(
            qn, kn, (((1,), (1,)), ((), ())),
            preferred_element_type=jnp.float32)
        outs = []
        for _ in range(TOP_K):
            m = jnp.max(s, axis=1, keepdims=True)
            cand = jnp.where(s == m, col, POOL)
            idx = jnp.min(cand, axis=1, keepdims=True)
            outs.append(idx)
            s = jnp.where(col == idx, neg, s)
        base = (pl.program_id(0) * LPS + j) * POOL
        idx_ref[j] = jnp.concatenate(outs, axis=1) + base


def _topk_indices(xq, nq, e_k, nk):
    return pl.pallas_call(
        _topk_body,
        grid=(N_LAYERS // LPS,),
        in_specs=[
            pl.BlockSpec((LPS, B, KEY_D), lambda l: (l, 0, 0)),
            pl.BlockSpec((LPS, B, 1), lambda l: (l, 0, 0)),
            pl.BlockSpec((LPS, POOL, KEY_D), lambda l: (l, 0, 0)),
            pl.BlockSpec((LPS, POOL, 1), lambda l: (l, 0, 0)),
        ],
        out_specs=pl.BlockSpec((LPS, B, TOP_K), lambda l: (l, 0, 0)),
        out_shape=jax.ShapeDtypeStruct((N_LAYERS, B, TOP_K), jnp.int32),
    )(xq, nq, e_k, nk)


def _sc_gather_body(table_hbm, idx_hbm, out_hbm, idx_v, buf0, buf1,
                    gsem0, gsem1, ssem0, ssem1):
    wid = lax.axis_index("s") * NUM_CORES + lax.axis_index("c")
    pltpu.sync_copy(idx_hbm.at[wid], idx_v)
    base = wid * RPW

    bufs = (buf0, buf1)
    gsems = (gsem0, gsem1)
    ssems = (ssem0, ssem1)

    def ostart(c):
        return out_hbm.at[pl.ds(base + c * CHUNK, CHUNK)]

    for c in range(NBUF - 1):
        pltpu.async_copy(table_hbm.at[idx_v.at[c]], bufs[c], gsems[c])

    def chunk(c, carry):
        for p in range(NBUF):
            @pl.when(c % NBUF == p)
            def _():
                pltpu.make_async_copy(table_hbm.at[idx_v.at[c]], bufs[p],
                                      gsems[p]).wait()
                pltpu.async_copy(bufs[p], ostart(c), ssems[p])

        nxt = c + NBUF - 1
        for p in range(NBUF):
            @pl.when(jnp.logical_and(nxt < NCHUNK, nxt % NBUF == p))
            def _():
                @pl.when(nxt >= NBUF)
                def _():
                    pltpu.make_async_copy(bufs[p], ostart(nxt - NBUF),
                                          ssems[p]).wait()
                pltpu.async_copy(table_hbm.at[idx_v.at[nxt]], bufs[p], gsems[p])

        return carry

    lax.fori_loop(0, NCHUNK, chunk, 0)
    for last in range(NCHUNK - NBUF, NCHUNK):
        pltpu.make_async_copy(bufs[last % NBUF], ostart(last),
                              ssems[last % NBUF]).wait()


@functools.lru_cache(maxsize=1)
def _sc_gather():
    return functools.partial(
        pl.kernel,
        mesh=plsc.VectorSubcoreMesh(core_axis_name="c", subcore_axis_name="s"),
        out_type=jax.ShapeDtypeStruct((ROWS, NUM_PROMPTS, EMB_D), jnp.float32),
        scratch_types=(
            [pltpu.VMEM((NCHUNK, CHUNK), jnp.int32)]
            + [pltpu.VMEM((CHUNK, NUM_PROMPTS, EMB_D), jnp.float32)] * NBUF
            + [pltpu.SemaphoreType.DMA] * (2 * NBUF)
        ),
        compiler_params=pltpu.CompilerParams(use_tc_tiling_on_sc=True),
    )(_sc_gather_body)


def kernel(x_query, e_k, e_p, vis_mark):
    xq = jnp.swapaxes(x_query, 0, 1)
    nq = jnp.maximum(jnp.linalg.norm(xq, axis=-1, keepdims=True), 1e-12)
    nk = jnp.maximum(jnp.linalg.norm(e_k, axis=-1, keepdims=True), 1e-12)
    idx = _topk_indices(xq, nq, e_k, nk)
    flat_idx = idx.reshape(NW, NCHUNK, CHUNK)
    table = e_p.reshape(N_LAYERS * POOL, NUM_PROMPTS, EMB_D)
    out = _sc_gather()(table, flat_idx)
    out = out.reshape(N_LAYERS, B, TOP_K * NUM_PROMPTS, EMB_D)
    p_loss = jnp.zeros((), jnp.float32)
    return (out, p_loss)


# speedup vs baseline: 1.1153x; 1.0028x over previous
import functools

import jax
import jax.numpy as jnp
from jax import lax
from jax.experimental import pallas as pl
from jax.experimental.pallas import tpu as pltpu
from jax.experimental.pallas import tpu_sc as plsc

B = 128
N_LAYERS = 12
KEY_D = 768
EMB_D = 768
POOL = 512
NUM_PROMPTS = 8
TOP_K = 5

ROW_D = NUM_PROMPTS * EMB_D
ROWS = N_LAYERS * B * TOP_K
NUM_CORES = 2
NUM_SUBCORES = 16
NW = NUM_CORES * NUM_SUBCORES
RPW = ROWS // NW
CHUNK = 10
NCHUNK = RPW // CHUNK
NBUF = 2
LPS = 6


def _topk_body(q_ref, nq_ref, k_ref, nk_ref, idx_ref):
    col = lax.broadcasted_iota(jnp.int32, (B, POOL), 1)
    neg = jnp.float32(-jnp.inf)
    for j in range(LPS):
        qn = q_ref[j] / nq_ref[j]
        kn = k_ref[j] / nk_ref[j]
        s = jax.lax.dot_general(
            qn, kn, (((1,), (1,)), ((), ())),
            preferred_element_type=jnp.float32)
        outs = []
        for _ in range(TOP_K):
            m = jnp.max(s, axis=1, keepdims=True)
            cand = jnp.where(s == m, col, POOL)
            idx = jnp.min(cand, axis=1, keepdims=True)
            outs.append(idx)
            s = jnp.where(col == idx, neg, s)
        base = (pl.program_id(0) * LPS + j) * POOL
        idx_ref[j] = jnp.concatenate(outs, axis=1) + base


def _topk_indices(xq, nq, e_k, nk):
    return pl.pallas_call(
        _topk_body,
        grid=(N_LAYERS // LPS,),
        in_specs=[
            pl.BlockSpec((LPS, B, KEY_D), lambda l: (l, 0, 0)),
            pl.BlockSpec((LPS, B, 1), lambda l: (l, 0, 0)),
            pl.BlockSpec((LPS, POOL, KEY_D), lambda l: (l, 0, 0)),
            pl.BlockSpec((LPS, POOL, 1), lambda l: (l, 0, 0)),
        ],
        out_specs=pl.BlockSpec((LPS, B, TOP_K), lambda l: (l, 0, 0)),
        out_shape=jax.ShapeDtypeStruct((N_LAYERS, B, TOP_K), jnp.int32),
    )(xq, nq, e_k, nk)


def _sc_gather_body(table_hbm, idx_hbm, out_hbm, idx_v, buf0, buf1,
                    gsem0, gsem1, ssem0, ssem1):
    wid = lax.axis_index("s") * NUM_CORES + lax.axis_index("c")
    pltpu.sync_copy(idx_hbm.at[wid], idx_v)
    base = wid * RPW

    bufs = (buf0, buf1)
    gsems = (gsem0, gsem1)
    ssems = (ssem0, ssem1)

    def ostart(c):
        return out_hbm.at[pl.ds(base + c * CHUNK, CHUNK)]

    for c in range(NBUF - 1):
        pltpu.async_copy(table_hbm.at[idx_v.at[c]], bufs[c], gsems[c])

    def chunk(c, carry):
        for p in range(NBUF):
            @pl.when(c % NBUF == p)
            def _():
                pltpu.make_async_copy(table_hbm.at[idx_v.at[c]], bufs[p],
                                      gsems[p]).wait()
                pltpu.async_copy(bufs[p], ostart(c), ssems[p])

        nxt = c + NBUF - 1
        for p in range(NBUF):
            @pl.when(jnp.logical_and(nxt < NCHUNK, nxt % NBUF == p))
            def _():
                @pl.when(nxt >= NBUF)
                def _():
                    pltpu.make_async_copy(bufs[p], ostart(nxt - NBUF),
                                          ssems[p]).wait()
                pltpu.async_copy(table_hbm.at[idx_v.at[nxt]], bufs[p], gsems[p])

        return carry

    lax.fori_loop(0, NCHUNK, chunk, 0)
    for last in range(NCHUNK - NBUF, NCHUNK):
        pltpu.make_async_copy(bufs[last % NBUF], ostart(last),
                              ssems[last % NBUF]).wait()


@functools.lru_cache(maxsize=1)
def _sc_gather():
    return functools.partial(
        pl.kernel,
        mesh=plsc.VectorSubcoreMesh(core_axis_name="c", subcore_axis_name="s"),
        out_type=jax.ShapeDtypeStruct((ROWS, NUM_PROMPTS, EMB_D), jnp.float32),
        scratch_types=(
            [pltpu.VMEM((NCHUNK, CHUNK), jnp.int32)]
            + [pltpu.VMEM((CHUNK, NUM_PROMPTS, EMB_D), jnp.float32)] * NBUF
            + [pltpu.SemaphoreType.DMA] * (2 * NBUF)
        ),
        compiler_params=pltpu.CompilerParams(use_tc_tiling_on_sc=True),
    )(_sc_gather_body)


def kernel(x_query, e_k, e_p, vis_mark):
    xq = jnp.swapaxes(x_query, 0, 1)
    nq = jnp.maximum(jnp.linalg.norm(xq, axis=-1, keepdims=True), 1e-12)
    nk = jnp.maximum(jnp.linalg.norm(e_k, axis=-1, keepdims=True), 1e-12)
    idx = _topk_indices(xq, nq, e_k, nk)
    flat_idx = idx.reshape(NW, NCHUNK, CHUNK)
    table = e_p.reshape(N_LAYERS * POOL, NUM_PROMPTS, EMB_D)
    out = _sc_gather()(table, flat_idx)
    out = out.reshape(N_LAYERS, B, TOP_K * NUM_PROMPTS, EMB_D)
    p_loss = jnp.zeros((), jnp.float32)
    return (out, p_loss)
